# baseline (speedup 1.0000x reference)
import functools

import jax
import jax.numpy as jnp
from jax import lax
from jax.experimental import pallas as pl
from jax.experimental.pallas import tpu as pltpu
import jax.experimental.pallas.tpu_sc as plsc

D_IN = 1024
D_HID = 2816
D_OUT = 1024
E = 8
K = 2
T = 4096
TILE = 256
P = T * K + E * TILE
NT = P // TILE
GATE_BLK = 512
NG = T // GATE_BLK
EPS = 1e-10
LOSS_W = 0.01


def _gate_body(x_ref, wg_ref, ltri_ref, utri_ref,
               e0_ref, e1_ref, r0_ref, r1_ref, w0_ref, w1_ref,
               imp_ref, load_ref, loss_ref, pad_off_ref, tile_e_ref):
    i = pl.program_id(0)

    @pl.when(i == 0)
    def _init():
        imp_ref[...] = jnp.zeros_like(imp_ref)
        load_ref[...] = jnp.zeros_like(load_ref)

    x = x_ref[...]
    wg = wg_ref[...]
    logits = jnp.dot(x.astype(jnp.bfloat16), wg.astype(jnp.bfloat16),
                     preferred_element_type=jnp.float32)
    iota_e = lax.broadcasted_iota(jnp.int32, (GATE_BLK, E), 1)
    m0 = jnp.max(logits, axis=1, keepdims=True)
    e0 = jnp.min(jnp.where(logits == m0, iota_e, E), axis=1, keepdims=True)
    l2 = jnp.where(iota_e == e0, -jnp.inf, logits)
    m1 = jnp.max(l2, axis=1, keepdims=True)
    e1 = jnp.min(jnp.where(l2 == m1, iota_e, E), axis=1, keepdims=True)
    z = jnp.exp(m1 - m0)
    w0 = 1.0 / (1.0 + z)
    w1 = z / (1.0 + z)

    oh0 = (iota_e == e0).astype(jnp.float32)
    oh1 = (iota_e == e1).astype(jnp.float32)
    c01 = oh0 + oh1
    excl = jnp.dot(ltri_ref[...], c01, preferred_element_type=jnp.float32,
                   precision=lax.Precision.HIGHEST)
    carry = load_ref[...]
    rank = excl + carry
    r0 = jnp.sum(rank * oh0, axis=1, keepdims=True)
    r1 = jnp.sum(rank * oh1, axis=1, keepdims=True)

    e0_ref[...] = e0.astype(jnp.int32).reshape(1, 1, GATE_BLK)
    e1_ref[...] = e1.astype(jnp.int32).reshape(1, 1, GATE_BLK)
    r0_ref[...] = r0.astype(jnp.int32).reshape(1, 1, GATE_BLK)
    r1_ref[...] = r1.astype(jnp.int32).reshape(1, 1, GATE_BLK)
    w0_ref[...] = w0.reshape(1, 1, GATE_BLK)
    w1_ref[...] = w1.reshape(1, 1, GATE_BLK)

    load_ref[...] = carry + jnp.sum(c01, axis=0, keepdims=True)
    imp_ref[...] = imp_ref[...] + jnp.sum(oh0 * w0 + oh1 * w1, axis=0,
                                          keepdims=True)

    @pl.when(i == NG - 1)
    def _final():
        counts = load_ref[...]
        ci = counts.astype(jnp.int32)
        pad = ((ci + (TILE - 1)) // TILE) * TILE
        incl = jnp.dot(pad.astype(jnp.float32), utri_ref[...],
                       preferred_element_type=jnp.float32,
                       precision=lax.Precision.HIGHEST).astype(jnp.int32)
        pad_off_ref[...] = incl - pad
        starts = lax.broadcasted_iota(jnp.int32, (1, NT), 1) * TILE
        te = jnp.sum((incl.reshape(E, 1) <= starts).astype(jnp.int32),
                     axis=0, keepdims=True)
        tile_e_ref[...] = jnp.minimum(te, E - 1)

        imp = imp_ref[...]
        ld = load_ref[...]

        def cv_sq(v):
            m = jnp.sum(v) / E
            var = jnp.sum((v - m) ** 2) / (E - 1)
            return var / (m * m + EPS)

        loss_ref[...] = ((cv_sq(imp) + cv_sq(ld)) * LOSS_W).reshape(1, 1)


def _run_gate(xt, Wg, interpret=False):
    ltri = jnp.tril(jnp.ones((GATE_BLK, GATE_BLK), jnp.float32), -1)
    utri = jnp.triu(jnp.ones((E, E), jnp.float32), 0)
    outs = pl.pallas_call(
        _gate_body,
        grid=(NG,),
        in_specs=[
            pl.BlockSpec((GATE_BLK, D_IN), lambda i: (i, 0)),
            pl.BlockSpec((D_IN, E), lambda i: (0, 0)),
            pl.BlockSpec((GATE_BLK, GATE_BLK), lambda i: (0, 0)),
            pl.BlockSpec((E, E), lambda i: (0, 0)),
        ],
        out_specs=[
            pl.BlockSpec((1, 1, GATE_BLK), lambda i: (i, 0, 0)),
            pl.BlockSpec((1, 1, GATE_BLK), lambda i: (i, 0, 0)),
            pl.BlockSpec((1, 1, GATE_BLK), lambda i: (i, 0, 0)),
            pl.BlockSpec((1, 1, GATE_BLK), lambda i: (i, 0, 0)),
            pl.BlockSpec((1, 1, GATE_BLK), lambda i: (i, 0, 0)),
            pl.BlockSpec((1, 1, GATE_BLK), lambda i: (i, 0, 0)),
            pl.BlockSpec((1, E), lambda i: (0, 0)),
            pl.BlockSpec((1, E), lambda i: (0, 0)),
            pl.BlockSpec((1, 1), lambda i: (0, 0)),
            pl.BlockSpec((1, E), lambda i: (0, 0)),
            pl.BlockSpec((1, NT), lambda i: (0, 0)),
        ],
        out_shape=[
            jax.ShapeDtypeStruct((NG, 1, GATE_BLK), jnp.int32),
            jax.ShapeDtypeStruct((NG, 1, GATE_BLK), jnp.int32),
            jax.ShapeDtypeStruct((NG, 1, GATE_BLK), jnp.int32),
            jax.ShapeDtypeStruct((NG, 1, GATE_BLK), jnp.int32),
            jax.ShapeDtypeStruct((NG, 1, GATE_BLK), jnp.float32),
            jax.ShapeDtypeStruct((NG, 1, GATE_BLK), jnp.float32),
            jax.ShapeDtypeStruct((1, E), jnp.float32),
            jax.ShapeDtypeStruct((1, E), jnp.float32),
            jax.ShapeDtypeStruct((1, 1), jnp.float32),
            jax.ShapeDtypeStruct((1, E), jnp.int32),
            jax.ShapeDtypeStruct((1, NT), jnp.int32),
        ],
        compiler_params=pltpu.CompilerParams(
            dimension_semantics=("arbitrary",)),
        interpret=interpret,
    )(xt, Wg, ltri, utri)
    return outs


NH = 2
HBLK = D_HID // NH


def _gemm_body_h0(te_ref, x_ref, wg_ref, bg_ref, wu_ref, bu_ref, wd_ref,
                  *rest):
    o_ref = rest[-1]
    xb = x_ref[...]
    g = jnp.dot(xb, wg_ref[0], preferred_element_type=jnp.float32)
    g = g + bg_ref[0]
    u = jnp.dot(xb, wu_ref[0], preferred_element_type=jnp.float32)
    u = u + bu_ref[0]
    h = (g * jax.nn.sigmoid(g)) * u
    o_ref[...] = jnp.dot(h, wd_ref[0], preferred_element_type=jnp.float32)


def _gemm_body_h1(te_ref, x_ref, wg_ref, bg_ref, wu_ref, bu_ref, wd_ref,
                  bd_ref, sw_ref, op_ref, *rest):
    o_ref = rest[-1]
    xb = x_ref[...]
    g = jnp.dot(xb, wg_ref[0], preferred_element_type=jnp.float32)
    g = g + bg_ref[0]
    u = jnp.dot(xb, wu_ref[0], preferred_element_type=jnp.float32)
    u = u + bu_ref[0]
    h = (g * jax.nn.sigmoid(g)) * u
    o = jnp.dot(h, wd_ref[0], preferred_element_type=jnp.float32)
    sw = sw_ref[0, 0, :][:, None]
    o_ref[...] = (op_ref[...] + o + bd_ref[0]) * sw


def _gemm_quarter(xs_half, o_prev, t0, nt, h, slot_w3, tile_expert,
                  W_gate_proj, b_gate_proj, W_up, b_up, W_down, b_down,
                  xdep=None, interpret=False):
    body = _gemm_body_h0 if h == 0 else _gemm_body_h1
    wsp = [
        pl.BlockSpec((1, D_IN, HBLK), lambda i, te: (te[t0 + i], 0, h)),
        pl.BlockSpec((1, 1, HBLK), lambda i, te: (te[t0 + i], 0, h)),
        pl.BlockSpec((1, D_IN, HBLK), lambda i, te: (te[t0 + i], 0, h)),
        pl.BlockSpec((1, 1, HBLK), lambda i, te: (te[t0 + i], 0, h)),
        pl.BlockSpec((1, HBLK, D_OUT), lambda i, te: (te[t0 + i], h, 0)),
    ]
    xspec = pl.BlockSpec((TILE, D_IN), lambda i, te: (i, 0))
    ospec = pl.BlockSpec((TILE, D_OUT), lambda i, te: (t0 + i, 0))
    cp = pltpu.CompilerParams(dimension_semantics=("arbitrary",),
                              vmem_limit_bytes=128 * 1024 * 1024)
    args = [tile_expert, xs_half, W_gate_proj, b_gate_proj, W_up, b_up,
            W_down]
    in_specs = [xspec] + wsp
    if h == 0:
        alias = {} if o_prev is None else {7: 0}
        if o_prev is not None:
            in_specs = in_specs + [pl.BlockSpec(memory_space=pl.ANY)]
            args = args + [o_prev]
    else:
        in_specs = in_specs + [
            pl.BlockSpec((1, 1, D_OUT), lambda i, te: (te[t0 + i], 0, 0)),
            pl.BlockSpec((1, 1, TILE), lambda i, te: (t0 + i, 0, 0)),
            ospec,
        ]
        args = args + [b_down, slot_w3, o_prev]
        alias = {9: 0}
    if xdep is not None:
        in_specs = in_specs + [pl.BlockSpec(memory_space=pl.ANY)]
        args = args + [xdep]
    gs = pltpu.PrefetchScalarGridSpec(
        num_scalar_prefetch=1, grid=(nt,),
        in_specs=in_specs, out_specs=ospec)
    return pl.pallas_call(
        body, grid_spec=gs,
        out_shape=jax.ShapeDtypeStruct((P, D_OUT), jnp.float32),
        input_output_aliases=alias,
        compiler_params=cp, interpret=interpret,
    )(*args)


_SC_INFO = None


def _sc_mesh():
    return plsc.VectorSubcoreMesh(core_axis_name="c", subcore_axis_name="s")


def _wid():
    info = plsc.get_sparse_core_info()
    return lax.axis_index("s") * info.num_cores + lax.axis_index("c")


def _run_routing(e0, e1, r0, r1, w0, w1, pad_off16):
    @functools.partial(
        pl.kernel,
        out_type=(
            jax.ShapeDtypeStruct((P,), jnp.int32),
            jax.ShapeDtypeStruct((P,), jnp.float32),
            jax.ShapeDtypeStruct((T,), jnp.int32),
            jax.ShapeDtypeStruct((T,), jnp.int32),
        ),
        mesh=_sc_mesh(),
        compiler_params=pltpu.CompilerParams(needs_layout_passes=False),
        scratch_types=[
            pltpu.VMEM((T,), jnp.int32),
            pltpu.VMEM((T,), jnp.int32),
            pltpu.VMEM((T,), jnp.float32),
            pltpu.VMEM((16,), jnp.int32),
            pltpu.VMEM((P,), jnp.int32),
            pltpu.VMEM((P,), jnp.float32),
            pltpu.VMEM((T,), jnp.int32),
        ],
    )
    def route(e0_h, e1_h, r0_h, r1_h, w0_h, w1_h, off_h,
              st_h, sw_h, d0_h, d1_h,
              e_v, r_v, w_v, off_v, st_v, sw_v, d_v):
        w = _wid()

        @pl.when(w == 0)
        def _():
            pltpu.sync_copy(off_h, off_v)

            def zero_body(j, _):
                st_v[pl.ds(j * 16, 16)] = jnp.zeros((16,), jnp.int32)
                sw_v[pl.ds(j * 16, 16)] = jnp.zeros((16,), jnp.float32)
                return 0

            lax.fori_loop(0, P // 16, zero_body, 0)

            def one_k(e_h, r_h, w_h, d_h):
                pltpu.sync_copy(e_h, e_v)
                pltpu.sync_copy(r_h, r_v)
                pltpu.sync_copy(w_h, w_v)

                def body(j, _):
                    sl = pl.ds(j * 16, 16)
                    idx = e_v[sl]
                    off = plsc.load_gather(off_v, [idx])
                    dest = off + r_v[sl]
                    tok = lax.iota(jnp.int32, 16) + j * 16
                    plsc.store_scatter(st_v, [dest], tok)
                    plsc.store_scatter(sw_v, [dest], w_v[sl])
                    d_v[sl] = dest
                    return 0

                lax.fori_loop(0, T // 16, body, 0)
                pltpu.sync_copy(d_v, d_h)

            one_k(e0_h, r0_h, w0_h, d0_h)
            one_k(e1_h, r1_h, w1_h, d1_h)
            pltpu.sync_copy(st_v, st_h)
            pltpu.sync_copy(sw_v, sw_h)

    return route(e0, e1, r0, r1, w0, w1, pad_off16)


_GCHUNK = 16
_GNBUF = 6
_NBUF = 3


def _run_gather(xt, slot_token, lo, nrows):
    info = plsc.get_sparse_core_info()
    nw = info.num_cores * info.num_subcores
    rows_per_w = nrows // nw
    nch = rows_per_w // _GCHUNK
    lead = 4

    @functools.partial(
        pl.kernel,
        out_type=jax.ShapeDtypeStruct((nrows, D_IN), jnp.float32),
        mesh=_sc_mesh(),
        compiler_params=pltpu.CompilerParams(needs_layout_passes=False),
        scratch_types=(
            [pltpu.VMEM((rows_per_w,), jnp.int32)]
            + [pltpu.VMEM((_GCHUNK, D_IN), jnp.float32)] * _GNBUF
            + [pltpu.SemaphoreType.DMA] * (2 * _GNBUF)
        ),
    )
    def gather(x_h, st_h, out_h, idx_all, b0, b1, b2, b3, b4, b5,
               g0, g1, g2, g3, g4, g5, w0, w1, w2, w3, w4, w5):
        base = _wid() * rows_per_w
        bufs = (b0, b1, b2, b3, b4, b5)
        gsem = (g0, g1, g2, g3, g4, g5)
        wsem = (w0, w1, w2, w3, w4, w5)
        pltpu.sync_copy(st_h.at[pl.ds(lo + base, rows_per_w)], idx_all)

        pg, pw = {}, {}

        def start_gather(c):
            b = c % _GNBUF
            pg[c] = pltpu.async_copy(
                x_h.at[idx_all.at[pl.ds(c * _GCHUNK, _GCHUNK)]],
                bufs[b], gsem[b])

        for c in range(min(lead, nch)):
            start_gather(c)
        for c in range(nch):
            b = c % _GNBUF
            pg[c].wait()
            pw[c] = pltpu.async_copy(
                bufs[b], out_h.at[pl.ds(base + c * _GCHUNK, _GCHUNK)],
                wsem[b])
            nxt = c + lead
            if nxt < nch:
                if nxt - _GNBUF >= 0:
                    pw[nxt - _GNBUF].wait()
                start_gather(nxt)
        for c in range(max(0, nch - _GNBUF), nch):
            pw[c].wait()

    return gather(xt, slot_token)


_CCHUNK = 16


def _run_combine(o_sorted, dest0, dest1):
    info = plsc.get_sparse_core_info()
    nw = info.num_cores * info.num_subcores
    tok_per_w = T // nw
    nch = tok_per_w // _CCHUNK
    nseg = D_OUT // 16

    @functools.partial(
        pl.kernel,
        out_type=jax.ShapeDtypeStruct((T, D_OUT), jnp.float32),
        mesh=_sc_mesh(),
        compiler_params=pltpu.CompilerParams(needs_layout_passes=False),
        scratch_types=(
            [pltpu.VMEM((tok_per_w,), jnp.int32)] * 2
            + [pltpu.VMEM((_CCHUNK, D_OUT), jnp.float32)] * (2 * _NBUF)
            + [pltpu.SemaphoreType.DMA] * (3 * _NBUF)
        ),
    )
    def combine(o_h, d0_h, d1_h, y_h, i0_all, i1_all,
                a0, a1, a2, c0, c1, c2,
                ga0, ga1, ga2, gb0, gb1, gb2, ws0, ws1, ws2):
        base = _wid() * tok_per_w
        abuf = (a0, a1, a2)
        bbuf = (c0, c1, c2)
        gsa = (ga0, ga1, ga2)
        gsb = (gb0, gb1, gb2)
        wsem = (ws0, ws1, ws2)
        pltpu.sync_copy(d0_h.at[pl.ds(base, tok_per_w)], i0_all)
        pltpu.sync_copy(d1_h.at[pl.ds(base, tok_per_w)], i1_all)

        pga, pgb, pw = {}, {}, {}

        def start(c):
            b = c % _NBUF
            sl = pl.ds(c * _CCHUNK, _CCHUNK)
            pga[c] = pltpu.async_copy(o_h.at[i0_all.at[sl]], abuf[b], gsa[b])
            pgb[c] = pltpu.async_copy(o_h.at[i1_all.at[sl]], bbuf[b], gsb[b])

        start(0)
        if nch > 1:
            start(1)
        for c in range(nch):
            b = c % _NBUF
            pga[c].wait()
            pgb[c].wait()
            A = abuf[b]
            Bv = bbuf[b]

            nq = nseg // 4

            def _add(j, _):
                row = j // nq
                s = (j % nq) * 4
                for k in range(4):
                    sl = pl.ds((s + k) * 16, 16)
                    A[row, sl] = A[row, sl] + Bv[row, sl]
                return 0

            lax.fori_loop(0, _CCHUNK * nq, _add, 0)

            pw[c] = pltpu.async_copy(
                A, y_h.at[pl.ds(base + c * _CCHUNK, _CCHUNK)], wsem[b])
            nxt = c + 2
            if nxt < nch:
                if nxt - _NBUF >= 0:
                    pw[nxt - _NBUF].wait()
                start(nxt)
        for c in range(max(0, nch - _NBUF), nch):
            pw[c].wait()

    return combine(o_sorted, dest0, dest1)


def kernel(x, Wg, W_gate_proj, b_gate_proj, W_up, b_up, W_down, b_down):
    B, S, _ = x.shape
    xt = x.reshape(T, D_IN)

    (e0, e1, r0, r1, w0, w1, imp, load, loss, pad_off,
     tile_expert) = _run_gate(xt, Wg)

    e0 = e0.reshape(T)
    e1 = e1.reshape(T)
    r0 = r0.reshape(T)
    r1 = r1.reshape(T)
    w0 = w0.reshape(T)
    w1 = w1.reshape(T)
    pad_off16 = jnp.pad(pad_off.reshape(E), (0, 8))
    tile_expert = tile_expert.reshape(NT)

    slot_token, slot_w, dest0, dest1 = _run_routing(
        e0, e1, r0, r1, w0, w1, pad_off16)

    HALF = P // 2
    NTH = NT // 2
    xsA = _run_gather(xt, slot_token, 0, HALF)
    xsB = _run_gather(xt, slot_token, HALF, HALF)

    slot_w3 = slot_w.reshape(NT, 1, TILE)
    b_gate_proj = b_gate_proj.reshape(E, 1, D_HID)
    b_up = b_up.reshape(E, 1, D_HID)
    b_down = b_down.reshape(E, 1, D_OUT)
    gq = functools.partial(
        _gemm_quarter, slot_w3=slot_w3, tile_expert=tile_expert,
        W_gate_proj=W_gate_proj, b_gate_proj=b_gate_proj, W_up=W_up,
        b_up=b_up, W_down=W_down, b_down=b_down)
    o1 = gq(xsA, None, 0, NTH, 0, xdep=xsB)
    o2 = gq(xsB, o1, NTH, NTH, 0)
    o3 = gq(xsA, o2, 0, NTH, 1)
    o_sorted = gq(xsB, o3, NTH, NTH, 1)

    y = _run_combine(o_sorted, dest0, dest1)

    return (y.reshape(B, S, D_OUT), loss.reshape(()))


# speedup vs baseline: 1.0563x; 1.0563x over previous
import functools

import jax
import jax.numpy as jnp
from jax import lax
from jax.experimental import pallas as pl
from jax.experimental.pallas import tpu as pltpu
import jax.experimental.pallas.tpu_sc as plsc

D_IN = 1024
D_HID = 2816
D_OUT = 1024
E = 8
K = 2
T = 4096
TILE = 256
P = T * K + E * TILE
NT = P // TILE
GATE_BLK = 512
NG = T // GATE_BLK
EPS = 1e-10
LOSS_W = 0.01


def _gate_body(x_ref, wg_ref, ltri_ref, utri_ref,
               e0_ref, e1_ref, r0_ref, r1_ref, w0_ref, w1_ref,
               imp_ref, load_ref, loss_ref, pad_off_ref, tile_e_ref):
    i = pl.program_id(0)

    @pl.when(i == 0)
    def _init():
        imp_ref[...] = jnp.zeros_like(imp_ref)
        load_ref[...] = jnp.zeros_like(load_ref)

    x = x_ref[...]
    wg = wg_ref[...]
    logits = jnp.dot(x.astype(jnp.bfloat16), wg.astype(jnp.bfloat16),
                     preferred_element_type=jnp.float32)
    iota_e = lax.broadcasted_iota(jnp.int32, (GATE_BLK, E), 1)
    m0 = jnp.max(logits, axis=1, keepdims=True)
    e0 = jnp.min(jnp.where(logits == m0, iota_e, E), axis=1, keepdims=True)
    l2 = jnp.where(iota_e == e0, -jnp.inf, logits)
    m1 = jnp.max(l2, axis=1, keepdims=True)
    e1 = jnp.min(jnp.where(l2 == m1, iota_e, E), axis=1, keepdims=True)
    z = jnp.exp(m1 - m0)
    w0 = 1.0 / (1.0 + z)
    w1 = z / (1.0 + z)

    oh0 = (iota_e == e0).astype(jnp.float32)
    oh1 = (iota_e == e1).astype(jnp.float32)
    c01 = oh0 + oh1
    excl = jnp.dot(ltri_ref[...], c01, preferred_element_type=jnp.float32,
                   precision=lax.Precision.HIGHEST)
    carry = load_ref[...]
    rank = excl + carry
    r0 = jnp.sum(rank * oh0, axis=1, keepdims=True)
    r1 = jnp.sum(rank * oh1, axis=1, keepdims=True)

    e0_ref[...] = e0.astype(jnp.int32).reshape(1, 1, GATE_BLK)
    e1_ref[...] = e1.astype(jnp.int32).reshape(1, 1, GATE_BLK)
    r0_ref[...] = r0.astype(jnp.int32).reshape(1, 1, GATE_BLK)
    r1_ref[...] = r1.astype(jnp.int32).reshape(1, 1, GATE_BLK)
    w0_ref[...] = w0.reshape(1, 1, GATE_BLK)
    w1_ref[...] = w1.reshape(1, 1, GATE_BLK)

    load_ref[...] = carry + jnp.sum(c01, axis=0, keepdims=True)
    imp_ref[...] = imp_ref[...] + jnp.sum(oh0 * w0 + oh1 * w1, axis=0,
                                          keepdims=True)

    @pl.when(i == NG - 1)
    def _final():
        counts = load_ref[...]
        ci = counts.astype(jnp.int32)
        pad = ((ci + (TILE - 1)) // TILE) * TILE
        incl = jnp.dot(pad.astype(jnp.float32), utri_ref[...],
                       preferred_element_type=jnp.float32,
                       precision=lax.Precision.HIGHEST).astype(jnp.int32)
        pad_off_ref[...] = incl - pad
        starts = lax.broadcasted_iota(jnp.int32, (1, NT), 1) * TILE
        te = jnp.sum((incl.reshape(E, 1) <= starts).astype(jnp.int32),
                     axis=0, keepdims=True)
        tile_e_ref[...] = jnp.minimum(te, E - 1)

        imp = imp_ref[...]
        ld = load_ref[...]

        def cv_sq(v):
            m = jnp.sum(v) / E
            var = jnp.sum((v - m) ** 2) / (E - 1)
            return var / (m * m + EPS)

        loss_ref[...] = ((cv_sq(imp) + cv_sq(ld)) * LOSS_W).reshape(1, 1)


def _run_gate(xt, Wg, interpret=False):
    ltri = jnp.tril(jnp.ones((GATE_BLK, GATE_BLK), jnp.float32), -1)
    utri = jnp.triu(jnp.ones((E, E), jnp.float32), 0)
    outs = pl.pallas_call(
        _gate_body,
        grid=(NG,),
        in_specs=[
            pl.BlockSpec((GATE_BLK, D_IN), lambda i: (i, 0)),
            pl.BlockSpec((D_IN, E), lambda i: (0, 0)),
            pl.BlockSpec((GATE_BLK, GATE_BLK), lambda i: (0, 0)),
            pl.BlockSpec((E, E), lambda i: (0, 0)),
        ],
        out_specs=[
            pl.BlockSpec((1, 1, GATE_BLK), lambda i: (i, 0, 0)),
            pl.BlockSpec((1, 1, GATE_BLK), lambda i: (i, 0, 0)),
            pl.BlockSpec((1, 1, GATE_BLK), lambda i: (i, 0, 0)),
            pl.BlockSpec((1, 1, GATE_BLK), lambda i: (i, 0, 0)),
            pl.BlockSpec((1, 1, GATE_BLK), lambda i: (i, 0, 0)),
            pl.BlockSpec((1, 1, GATE_BLK), lambda i: (i, 0, 0)),
            pl.BlockSpec((1, E), lambda i: (0, 0)),
            pl.BlockSpec((1, E), lambda i: (0, 0)),
            pl.BlockSpec((1, 1), lambda i: (0, 0)),
            pl.BlockSpec((1, E), lambda i: (0, 0)),
            pl.BlockSpec((1, NT), lambda i: (0, 0)),
        ],
        out_shape=[
            jax.ShapeDtypeStruct((NG, 1, GATE_BLK), jnp.int32),
            jax.ShapeDtypeStruct((NG, 1, GATE_BLK), jnp.int32),
            jax.ShapeDtypeStruct((NG, 1, GATE_BLK), jnp.int32),
            jax.ShapeDtypeStruct((NG, 1, GATE_BLK), jnp.int32),
            jax.ShapeDtypeStruct((NG, 1, GATE_BLK), jnp.float32),
            jax.ShapeDtypeStruct((NG, 1, GATE_BLK), jnp.float32),
            jax.ShapeDtypeStruct((1, E), jnp.float32),
            jax.ShapeDtypeStruct((1, E), jnp.float32),
            jax.ShapeDtypeStruct((1, 1), jnp.float32),
            jax.ShapeDtypeStruct((1, E), jnp.int32),
            jax.ShapeDtypeStruct((1, NT), jnp.int32),
        ],
        compiler_params=pltpu.CompilerParams(
            dimension_semantics=("arbitrary",)),
        interpret=interpret,
    )(xt, Wg, ltri, utri)
    return outs


NH = 2
HBLK = D_HID // NH


def _gemm_body_h0(te_ref, x_ref, wg_ref, bg_ref, wu_ref, bu_ref, wd_ref,
                  *rest):
    o_ref = rest[-1]
    xb = x_ref[...]
    g = jnp.dot(xb, wg_ref[0], preferred_element_type=jnp.float32)
    g = g + bg_ref[0]
    u = jnp.dot(xb, wu_ref[0], preferred_element_type=jnp.float32)
    u = u + bu_ref[0]
    h = (g * jax.nn.sigmoid(g)) * u
    o_ref[...] = jnp.dot(h, wd_ref[0], preferred_element_type=jnp.float32)


def _gemm_body_h1(te_ref, x_ref, wg_ref, bg_ref, wu_ref, bu_ref, wd_ref,
                  bd_ref, sw_ref, op_ref, o_ref):
    xb = x_ref[...]
    g = jnp.dot(xb, wg_ref[0], preferred_element_type=jnp.float32)
    g = g + bg_ref[0]
    u = jnp.dot(xb, wu_ref[0], preferred_element_type=jnp.float32)
    u = u + bu_ref[0]
    h = (g * jax.nn.sigmoid(g)) * u
    o = jnp.dot(h, wd_ref[0], preferred_element_type=jnp.float32)
    sw = sw_ref[0, 0, :][:, None]
    o_ref[...] = (op_ref[...] + o + bd_ref[0]) * sw


def _gemm_quarter(xs_half, o_prev, t0, nt, h, slot_w3, tile_expert,
                  W_gate_proj, b_gate_proj, W_up, b_up, W_down, b_down,
                  interpret=False):
    body = _gemm_body_h0 if h == 0 else _gemm_body_h1
    wsp = [
        pl.BlockSpec((1, D_IN, HBLK), lambda i, te: (te[t0 + i], 0, h)),
        pl.BlockSpec((1, 1, HBLK), lambda i, te: (te[t0 + i], 0, h)),
        pl.BlockSpec((1, D_IN, HBLK), lambda i, te: (te[t0 + i], 0, h)),
        pl.BlockSpec((1, 1, HBLK), lambda i, te: (te[t0 + i], 0, h)),
        pl.BlockSpec((1, HBLK, D_OUT), lambda i, te: (te[t0 + i], h, 0)),
    ]
    xspec = pl.BlockSpec((TILE, D_IN), lambda i, te: (i, 0))
    ospec = pl.BlockSpec((TILE, D_OUT), lambda i, te: (t0 + i, 0))
    cp = pltpu.CompilerParams(dimension_semantics=("arbitrary",),
                              vmem_limit_bytes=128 * 1024 * 1024)
    args = [tile_expert, xs_half, W_gate_proj, b_gate_proj, W_up, b_up,
            W_down]
    in_specs = [xspec] + wsp
    if h == 0:
        alias = {} if o_prev is None else {7: 0}
        if o_prev is not None:
            in_specs = in_specs + [ospec]
            args = args + [o_prev]
    else:
        in_specs = in_specs + [
            pl.BlockSpec((1, 1, D_OUT), lambda i, te: (te[t0 + i], 0, 0)),
            pl.BlockSpec((1, 1, TILE), lambda i, te: (t0 + i, 0, 0)),
            ospec,
        ]
        args = args + [b_down, slot_w3, o_prev]
        alias = {9: 0}
    gs = pltpu.PrefetchScalarGridSpec(
        num_scalar_prefetch=1, grid=(nt,),
        in_specs=in_specs, out_specs=ospec)
    return pl.pallas_call(
        body, grid_spec=gs,
        out_shape=jax.ShapeDtypeStruct((P, D_OUT), jnp.float32),
        input_output_aliases=alias,
        compiler_params=cp, interpret=interpret,
    )(*args)


_SC_INFO = None


def _sc_mesh():
    return plsc.VectorSubcoreMesh(core_axis_name="c", subcore_axis_name="s")


def _wid():
    info = plsc.get_sparse_core_info()
    return lax.axis_index("s") * info.num_cores + lax.axis_index("c")


def _run_routing(e0, e1, r0, r1, w0, w1, pad_off16):
    @functools.partial(
        pl.kernel,
        out_type=(
            jax.ShapeDtypeStruct((P,), jnp.int32),
            jax.ShapeDtypeStruct((P,), jnp.float32),
            jax.ShapeDtypeStruct((T,), jnp.int32),
            jax.ShapeDtypeStruct((T,), jnp.int32),
        ),
        mesh=_sc_mesh(),
        compiler_params=pltpu.CompilerParams(needs_layout_passes=False),
        scratch_types=[
            pltpu.VMEM((T,), jnp.int32),
            pltpu.VMEM((T,), jnp.int32),
            pltpu.VMEM((T,), jnp.float32),
            pltpu.VMEM((16,), jnp.int32),
            pltpu.VMEM((P,), jnp.int32),
            pltpu.VMEM((P,), jnp.float32),
            pltpu.VMEM((T,), jnp.int32),
        ],
    )
    def route(e0_h, e1_h, r0_h, r1_h, w0_h, w1_h, off_h,
              st_h, sw_h, d0_h, d1_h,
              e_v, r_v, w_v, off_v, st_v, sw_v, d_v):
        w = _wid()

        @pl.when(w == 0)
        def _():
            pltpu.sync_copy(off_h, off_v)

            def zero_body(j, _):
                st_v[pl.ds(j * 16, 16)] = jnp.zeros((16,), jnp.int32)
                sw_v[pl.ds(j * 16, 16)] = jnp.zeros((16,), jnp.float32)
                return 0

            lax.fori_loop(0, P // 16, zero_body, 0)

            def one_k(e_h, r_h, w_h, d_h):
                pltpu.sync_copy(e_h, e_v)
                pltpu.sync_copy(r_h, r_v)
                pltpu.sync_copy(w_h, w_v)

                def body(j, _):
                    sl = pl.ds(j * 16, 16)
                    idx = e_v[sl]
                    off = plsc.load_gather(off_v, [idx])
                    dest = off + r_v[sl]
                    tok = lax.iota(jnp.int32, 16) + j * 16
                    plsc.store_scatter(st_v, [dest], tok)
                    plsc.store_scatter(sw_v, [dest], w_v[sl])
                    d_v[sl] = dest
                    return 0

                lax.fori_loop(0, T // 16, body, 0)
                pltpu.sync_copy(d_v, d_h)

            one_k(e0_h, r0_h, w0_h, d0_h)
            one_k(e1_h, r1_h, w1_h, d1_h)
            pltpu.sync_copy(st_v, st_h)
            pltpu.sync_copy(sw_v, sw_h)

    return route(e0, e1, r0, r1, w0, w1, pad_off16)


_GCHUNK = 16
_GNBUF = 6
_NBUF = 3


def _run_gather(xt, slot_token, lo, nrows):
    info = plsc.get_sparse_core_info()
    nw = info.num_cores * info.num_subcores
    rows_per_w = nrows // nw
    nch = rows_per_w // _GCHUNK
    lead = 4

    @functools.partial(
        pl.kernel,
        out_type=jax.ShapeDtypeStruct((nrows, D_IN), jnp.float32),
        mesh=_sc_mesh(),
        compiler_params=pltpu.CompilerParams(needs_layout_passes=False),
        scratch_types=(
            [pltpu.VMEM((rows_per_w,), jnp.int32)]
            + [pltpu.VMEM((_GCHUNK, D_IN), jnp.float32)] * _GNBUF
            + [pltpu.SemaphoreType.DMA] * (2 * _GNBUF)
        ),
    )
    def gather(x_h, st_h, out_h, idx_all, b0, b1, b2, b3, b4, b5,
               g0, g1, g2, g3, g4, g5, w0, w1, w2, w3, w4, w5):
        base = _wid() * rows_per_w
        bufs = (b0, b1, b2, b3, b4, b5)
        gsem = (g0, g1, g2, g3, g4, g5)
        wsem = (w0, w1, w2, w3, w4, w5)
        pltpu.sync_copy(st_h.at[pl.ds(lo + base, rows_per_w)], idx_all)

        pg, pw = {}, {}

        def start_gather(c):
            b = c % _GNBUF
            pg[c] = pltpu.async_copy(
                x_h.at[idx_all.at[pl.ds(c * _GCHUNK, _GCHUNK)]],
                bufs[b], gsem[b])

        for c in range(min(lead, nch)):
            start_gather(c)
        for c in range(nch):
            b = c % _GNBUF
            pg[c].wait()
            pw[c] = pltpu.async_copy(
                bufs[b], out_h.at[pl.ds(base + c * _GCHUNK, _GCHUNK)],
                wsem[b])
            nxt = c + lead
            if nxt < nch:
                if nxt - _GNBUF >= 0:
                    pw[nxt - _GNBUF].wait()
                start_gather(nxt)
        for c in range(max(0, nch - _GNBUF), nch):
            pw[c].wait()

    return gather(xt, slot_token)


_CCHUNK = 16


def _run_combine(o_sorted, dest0, dest1):
    info = plsc.get_sparse_core_info()
    nw = info.num_cores * info.num_subcores
    tok_per_w = T // nw
    nch = tok_per_w // _CCHUNK
    nseg = D_OUT // 16

    @functools.partial(
        pl.kernel,
        out_type=jax.ShapeDtypeStruct((T, D_OUT), jnp.float32),
        mesh=_sc_mesh(),
        compiler_params=pltpu.CompilerParams(needs_layout_passes=False),
        scratch_types=(
            [pltpu.VMEM((tok_per_w,), jnp.int32)] * 2
            + [pltpu.VMEM((_CCHUNK, D_OUT), jnp.float32)] * (2 * _NBUF)
            + [pltpu.SemaphoreType.DMA] * (3 * _NBUF)
        ),
    )
    def combine(o_h, d0_h, d1_h, y_h, i0_all, i1_all,
                a0, a1, a2, c0, c1, c2,
                ga0, ga1, ga2, gb0, gb1, gb2, ws0, ws1, ws2):
        base = _wid() * tok_per_w
        abuf = (a0, a1, a2)
        bbuf = (c0, c1, c2)
        gsa = (ga0, ga1, ga2)
        gsb = (gb0, gb1, gb2)
        wsem = (ws0, ws1, ws2)
        pltpu.sync_copy(d0_h.at[pl.ds(base, tok_per_w)], i0_all)
        pltpu.sync_copy(d1_h.at[pl.ds(base, tok_per_w)], i1_all)

        pga, pgb, pw = {}, {}, {}

        def start(c):
            b = c % _NBUF
            sl = pl.ds(c * _CCHUNK, _CCHUNK)
            pga[c] = pltpu.async_copy(o_h.at[i0_all.at[sl]], abuf[b], gsa[b])
            pgb[c] = pltpu.async_copy(o_h.at[i1_all.at[sl]], bbuf[b], gsb[b])

        start(0)
        if nch > 1:
            start(1)
        for c in range(nch):
            b = c % _NBUF
            pga[c].wait()
            pgb[c].wait()
            A = abuf[b]
            Bv = bbuf[b]

            nq = nseg // 4

            def _add(j, _):
                row = j // nq
                s = (j % nq) * 4
                for k in range(4):
                    sl = pl.ds((s + k) * 16, 16)
                    A[row, sl] = A[row, sl] + Bv[row, sl]
                return 0

            lax.fori_loop(0, _CCHUNK * nq, _add, 0)

            pw[c] = pltpu.async_copy(
                A, y_h.at[pl.ds(base + c * _CCHUNK, _CCHUNK)], wsem[b])
            nxt = c + 2
            if nxt < nch:
                if nxt - _NBUF >= 0:
                    pw[nxt - _NBUF].wait()
                start(nxt)
        for c in range(max(0, nch - _NBUF), nch):
            pw[c].wait()

    return combine(o_sorted, dest0, dest1)


def kernel(x, Wg, W_gate_proj, b_gate_proj, W_up, b_up, W_down, b_down):
    B, S, _ = x.shape
    xt = x.reshape(T, D_IN)

    (e0, e1, r0, r1, w0, w1, imp, load, loss, pad_off,
     tile_expert) = _run_gate(xt, Wg)

    e0 = e0.reshape(T)
    e1 = e1.reshape(T)
    r0 = r0.reshape(T)
    r1 = r1.reshape(T)
    w0 = w0.reshape(T)
    w1 = w1.reshape(T)
    pad_off16 = jnp.pad(pad_off.reshape(E), (0, 8))
    tile_expert = tile_expert.reshape(NT)

    slot_token, slot_w, dest0, dest1 = _run_routing(
        e0, e1, r0, r1, w0, w1, pad_off16)

    HALF = P // 2
    NTH = NT // 2
    xsA = _run_gather(xt, slot_token, 0, HALF)
    xsB = _run_gather(xt, slot_token, HALF, HALF)

    slot_w3 = slot_w.reshape(NT, 1, TILE)
    b_gate_proj = b_gate_proj.reshape(E, 1, D_HID)
    b_up = b_up.reshape(E, 1, D_HID)
    b_down = b_down.reshape(E, 1, D_OUT)
    gq = functools.partial(
        _gemm_quarter, slot_w3=slot_w3, tile_expert=tile_expert,
        W_gate_proj=W_gate_proj, b_gate_proj=b_gate_proj, W_up=W_up,
        b_up=b_up, W_down=W_down, b_down=b_down)
    o1 = gq(xsA, None, 0, NTH, 0)
    o2 = gq(xsB, o1, NTH, NTH, 0)
    o3 = gq(xsA, o2, 0, NTH, 1)
    o_sorted = gq(xsB, o3, NTH, NTH, 1)

    y = _run_combine(o_sorted, dest0, dest1)

    return (y.reshape(B, S, D_OUT), loss.reshape(()))
